# baseline (device time: 412117 ns/iter reference)
import jax
import jax.numpy as jnp
from jax import lax
from jax.experimental import pallas as pl
from jax.experimental.pallas import tpu as pltpu

N_DEV = 16
S_LOC = 256
BH = 16
D = 64
SCALE = D ** -0.5


def _body(q_ref, kv_ref, o_ref, kv_all, send_sems, recv_sems):
    my = lax.axis_index("i")
    left = lax.rem(my + N_DEV - 1, N_DEV)
    right = lax.rem(my + 1, N_DEV)

    barrier_sem = pltpu.get_barrier_semaphore()
    for nbr in (left, right):
        pl.semaphore_signal(
            barrier_sem, inc=1,
            device_id=(nbr,), device_id_type=pl.DeviceIdType.MESH,
        )
    pl.semaphore_wait(barrier_sem, 2)

    kv_all[0] = kv_ref[...]

    rdmas = []
    for h in range(N_DEV - 1):
        src = kv_ref.at[...] if h == 0 else kv_all.at[h]
        rdma = pltpu.make_async_remote_copy(
            src_ref=src,
            dst_ref=kv_all.at[h + 1],
            send_sem=send_sems.at[h],
            recv_sem=recv_sems.at[h],
            device_id=(right,),
            device_id_type=pl.DeviceIdType.MESH,
        )
        rdma.start()
        rdma.wait_recv()
        rdmas.append(rdma)

    for bh in range(BH):
        q = q_ref[bh]
        k = jnp.reshape(kv_all[:, 0, bh], (N_DEV * S_LOC, D))
        s = lax.dot_general(
            q, k, (((1,), (1,)), ((), ())),
            preferred_element_type=jnp.float32,
        )
        m = jnp.max(s, axis=-1, keepdims=True)
        p = jnp.exp(s - m)
        denom = jnp.sum(p, axis=-1, keepdims=True)
        p = (p / denom).astype(jnp.bfloat16)
        v = jnp.reshape(kv_all[:, 1, bh], (N_DEV * S_LOC, D))
        o_ref[bh] = jnp.dot(p, v, preferred_element_type=jnp.float32)

    for rdma in rdmas:
        rdma.wait_send()


def kernel(Q, K, V):
    qt = (jnp.transpose(Q, (0, 2, 1, 3)).reshape(BH, S_LOC, D) * SCALE).astype(
        jnp.bfloat16
    )
    kt = jnp.transpose(K, (0, 2, 1, 3)).reshape(BH, S_LOC, D).astype(jnp.bfloat16)
    vt = jnp.transpose(V, (0, 2, 1, 3)).reshape(BH, S_LOC, D).astype(jnp.bfloat16)
    kv = jnp.stack([kt, vt])

    out = pl.pallas_call(
        _body,
        out_shape=jax.ShapeDtypeStruct((BH, S_LOC, D), jnp.float32),
        in_specs=[
            pl.BlockSpec(memory_space=pltpu.VMEM),
            pl.BlockSpec(memory_space=pltpu.VMEM),
        ],
        out_specs=pl.BlockSpec(memory_space=pltpu.VMEM),
        scratch_shapes=[
            pltpu.VMEM((N_DEV, 2, BH, S_LOC, D), jnp.bfloat16),
            pltpu.SemaphoreType.DMA((N_DEV - 1,)),
            pltpu.SemaphoreType.DMA((N_DEV - 1,)),
        ],
        compiler_params=pltpu.CompilerParams(
            collective_id=0, vmem_limit_bytes=100 * 1024 * 1024
        ),
    )(qt, kv)

    return jnp.transpose(out.reshape(2, 8, S_LOC, D), (0, 2, 1, 3))


# device time: 242889 ns/iter; 1.6967x vs baseline; 1.6967x over previous
import jax
import jax.numpy as jnp
from jax import lax
from jax.experimental import pallas as pl
from jax.experimental.pallas import tpu as pltpu

N_DEV = 16
CW_HOPS = 8
CCW_HOPS = 7
S_LOC = 256
BH = 16
D = 64
SCALE = D ** -0.5


def _body(q_ref, kv_ref, o_ref, kv_all,
          cw_send_sems, cw_recv_sems, ccw_send_sems, ccw_recv_sems):
    my = lax.axis_index("i")
    left = lax.rem(my + N_DEV - 1, N_DEV)
    right = lax.rem(my + 1, N_DEV)

    barrier_sem = pltpu.get_barrier_semaphore()
    for nbr in (left, right):
        pl.semaphore_signal(
            barrier_sem, inc=1,
            device_id=(nbr,), device_id_type=pl.DeviceIdType.MESH,
        )
    pl.semaphore_wait(barrier_sem, 2)

    kv_all[0] = kv_ref[...]

    def make_cw(h):
        return pltpu.make_async_remote_copy(
            src_ref=kv_ref.at[...] if h == 0 else kv_all.at[h],
            dst_ref=kv_all.at[h + 1],
            send_sem=cw_send_sems.at[h],
            recv_sem=cw_recv_sems.at[h],
            device_id=(right,),
            device_id_type=pl.DeviceIdType.MESH,
        )

    def make_ccw(g):
        return pltpu.make_async_remote_copy(
            src_ref=kv_ref.at[...] if g == 0 else kv_all.at[16 - g],
            dst_ref=kv_all.at[15 - g],
            send_sem=ccw_send_sems.at[g],
            recv_sem=ccw_recv_sems.at[g],
            device_id=(left,),
            device_id_type=pl.DeviceIdType.MESH,
        )

    cw = [make_cw(h) for h in range(CW_HOPS)]
    ccw = [make_ccw(g) for g in range(CCW_HOPS)]

    cw[0].start()
    ccw[0].start()
    for step in range(1, CW_HOPS):
        cw[step - 1].wait_recv()
        cw[step].start()
        if step < CCW_HOPS:
            ccw[step - 1].wait_recv()
            ccw[step].start()
    cw[CW_HOPS - 1].wait_recv()
    ccw[CCW_HOPS - 1].wait_recv()

    for bh in range(BH):
        q = q_ref[bh]
        k = jnp.reshape(kv_all[:, 0, bh], (N_DEV * S_LOC, D))
        s = lax.dot_general(
            q, k, (((1,), (1,)), ((), ())),
            preferred_element_type=jnp.float32,
        )
        m = jnp.max(s, axis=-1, keepdims=True)
        p = jnp.exp(s - m)
        denom = jnp.sum(p, axis=-1, keepdims=True)
        p = (p / denom).astype(jnp.bfloat16)
        v = jnp.reshape(kv_all[:, 1, bh], (N_DEV * S_LOC, D))
        o_ref[bh] = jnp.dot(p, v, preferred_element_type=jnp.float32)

    for rdma in cw + ccw:
        rdma.wait_send()


def kernel(Q, K, V):
    qt = (jnp.transpose(Q, (0, 2, 1, 3)).reshape(BH, S_LOC, D) * SCALE).astype(
        jnp.bfloat16
    )
    kt = jnp.transpose(K, (0, 2, 1, 3)).reshape(BH, S_LOC, D).astype(jnp.bfloat16)
    vt = jnp.transpose(V, (0, 2, 1, 3)).reshape(BH, S_LOC, D).astype(jnp.bfloat16)
    kv = jnp.stack([kt, vt])

    out = pl.pallas_call(
        _body,
        out_shape=jax.ShapeDtypeStruct((BH, S_LOC, D), jnp.float32),
        in_specs=[
            pl.BlockSpec(memory_space=pltpu.VMEM),
            pl.BlockSpec(memory_space=pltpu.VMEM),
        ],
        out_specs=pl.BlockSpec(memory_space=pltpu.VMEM),
        scratch_shapes=[
            pltpu.VMEM((N_DEV, 2, BH, S_LOC, D), jnp.bfloat16),
            pltpu.SemaphoreType.DMA((CW_HOPS,)),
            pltpu.SemaphoreType.DMA((CW_HOPS,)),
            pltpu.SemaphoreType.DMA((CCW_HOPS,)),
            pltpu.SemaphoreType.DMA((CCW_HOPS,)),
        ],
        compiler_params=pltpu.CompilerParams(
            collective_id=0, vmem_limit_bytes=100 * 1024 * 1024
        ),
    )(qt, kv)

    return jnp.transpose(out.reshape(2, 8, S_LOC, D), (0, 2, 1, 3))


# device time: 220685 ns/iter; 1.8674x vs baseline; 1.1006x over previous
import jax
import jax.numpy as jnp
from jax import lax
from jax.experimental import pallas as pl
from jax.experimental.pallas import tpu as pltpu

N_DEV = 16
CW_HOPS = 8
CCW_HOPS = 7
S_LOC = 256
BH = 16
D = 64
SCALE = D ** -0.5


def _body(q_ref, kv_ref, o_ref, kv_all, acc_ref, m_ref, l_ref,
          cw_send_sems, cw_recv_sems, ccw_send_sems, ccw_recv_sems):
    my = lax.axis_index("i")
    left = lax.rem(my + N_DEV - 1, N_DEV)
    right = lax.rem(my + 1, N_DEV)

    barrier_sem = pltpu.get_barrier_semaphore()
    for nbr in (left, right):
        pl.semaphore_signal(
            barrier_sem, inc=1,
            device_id=(nbr,), device_id_type=pl.DeviceIdType.MESH,
        )
    pl.semaphore_wait(barrier_sem, 2)

    def make_cw(h):
        return pltpu.make_async_remote_copy(
            src_ref=kv_ref.at[...] if h == 0 else kv_all.at[h],
            dst_ref=kv_all.at[h + 1],
            send_sem=cw_send_sems.at[h],
            recv_sem=cw_recv_sems.at[h],
            device_id=(right,),
            device_id_type=pl.DeviceIdType.MESH,
        )

    def make_ccw(g):
        return pltpu.make_async_remote_copy(
            src_ref=kv_ref.at[...] if g == 0 else kv_all.at[16 - g],
            dst_ref=kv_all.at[15 - g],
            send_sem=ccw_send_sems.at[g],
            recv_sem=ccw_recv_sems.at[g],
            device_id=(left,),
            device_id_type=pl.DeviceIdType.MESH,
        )

    cw = [make_cw(h) for h in range(CW_HOPS)]
    ccw = [make_ccw(g) for g in range(CCW_HOPS)]

    cw[0].start()
    ccw[0].start()
    kv_all[0] = kv_ref[...]

    def process(slot, first):
        def bh_step(bh, _):
            q = q_ref[bh]
            k = kv_all[slot, 0, bh]
            v = kv_all[slot, 1, bh]
            s = lax.dot_general(
                q, k, (((1,), (1,)), ((), ())),
                preferred_element_type=jnp.float32,
            )
            m_blk = jnp.max(s, axis=-1, keepdims=True)
            if first:
                m_new = m_blk
                p = jnp.exp(s - m_new)
                l_ref[bh] = jnp.sum(p, axis=-1, keepdims=True)
                acc_ref[bh] = jnp.dot(
                    p.astype(jnp.bfloat16), v,
                    preferred_element_type=jnp.float32,
                )
            else:
                m_old = m_ref[bh]
                m_new = jnp.maximum(m_old, m_blk)
                alpha = jnp.exp(m_old - m_new)
                p = jnp.exp(s - m_new)
                l_ref[bh] = l_ref[bh] * alpha + jnp.sum(
                    p, axis=-1, keepdims=True
                )
                acc_ref[bh] = acc_ref[bh] * alpha + jnp.dot(
                    p.astype(jnp.bfloat16), v,
                    preferred_element_type=jnp.float32,
                )
            m_ref[bh] = m_new
            return _

        lax.fori_loop(0, BH, bh_step, 0)

    process(0, first=True)

    for step in range(1, CW_HOPS + 1):
        cw[step - 1].wait_recv()
        if step < CW_HOPS:
            cw[step].start()
        have_ccw = step <= CCW_HOPS
        if have_ccw:
            ccw[step - 1].wait_recv()
            if step < CCW_HOPS:
                ccw[step].start()
        process(step, first=False)
        if have_ccw:
            process(16 - step, first=False)

    for bh in range(BH):
        o_ref[bh] = acc_ref[bh] / l_ref[bh]

    for rdma in cw + ccw:
        rdma.wait_send()


def kernel(Q, K, V):
    qt = (jnp.transpose(Q, (0, 2, 1, 3)).reshape(BH, S_LOC, D) * SCALE).astype(
        jnp.bfloat16
    )
    kt = jnp.transpose(K, (0, 2, 1, 3)).reshape(BH, S_LOC, D).astype(jnp.bfloat16)
    vt = jnp.transpose(V, (0, 2, 1, 3)).reshape(BH, S_LOC, D).astype(jnp.bfloat16)
    kv = jnp.stack([kt, vt])

    out = pl.pallas_call(
        _body,
        out_shape=jax.ShapeDtypeStruct((BH, S_LOC, D), jnp.float32),
        in_specs=[
            pl.BlockSpec(memory_space=pltpu.VMEM),
            pl.BlockSpec(memory_space=pltpu.VMEM),
        ],
        out_specs=pl.BlockSpec(memory_space=pltpu.VMEM),
        scratch_shapes=[
            pltpu.VMEM((N_DEV, 2, BH, S_LOC, D), jnp.bfloat16),
            pltpu.VMEM((BH, S_LOC, D), jnp.float32),
            pltpu.VMEM((BH, S_LOC, 1), jnp.float32),
            pltpu.VMEM((BH, S_LOC, 1), jnp.float32),
            pltpu.SemaphoreType.DMA((CW_HOPS,)),
            pltpu.SemaphoreType.DMA((CW_HOPS,)),
            pltpu.SemaphoreType.DMA((CCW_HOPS,)),
            pltpu.SemaphoreType.DMA((CCW_HOPS,)),
        ],
        compiler_params=pltpu.CompilerParams(
            collective_id=0, vmem_limit_bytes=100 * 1024 * 1024
        ),
    )(qt, kv)

    return jnp.transpose(out.reshape(2, 8, S_LOC, D), (0, 2, 1, 3))


# device time: 210092 ns/iter; 1.9616x vs baseline; 1.0504x over previous
import jax
import jax.numpy as jnp
from jax import lax
from jax.experimental import pallas as pl
from jax.experimental.pallas import tpu as pltpu

N_DEV = 16
CW_HOPS = 8
CCW_HOPS = 7
S_LOC = 256
BH = 16
D = 64
SCALE = D ** -0.5


def _body(q_ref, kv_ref, o_ref, kv_all, acc_ref, m_ref, l_ref,
          cw_send_sems, cw_recv_sems, ccw_send_sems, ccw_recv_sems):
    my = lax.axis_index("i")
    left = lax.rem(my + N_DEV - 1, N_DEV)
    right = lax.rem(my + 1, N_DEV)

    barrier_sem = pltpu.get_barrier_semaphore()
    for nbr in (left, right):
        pl.semaphore_signal(
            barrier_sem, inc=1,
            device_id=(nbr,), device_id_type=pl.DeviceIdType.MESH,
        )
    pl.semaphore_wait(barrier_sem, 2)

    HB = BH // 2

    def half(ref, part):
        lo = part * HB
        return ref.at[:, lo:lo + HB]

    def make_cw(h, part):
        src = kv_ref.at[...] if h == 0 else kv_all.at[h]
        return pltpu.make_async_remote_copy(
            src_ref=half(src, part),
            dst_ref=half(kv_all.at[h + 1], part),
            send_sem=cw_send_sems.at[h, part],
            recv_sem=cw_recv_sems.at[h, part],
            device_id=(right,),
            device_id_type=pl.DeviceIdType.MESH,
        )

    def make_ccw(g, part):
        src = kv_ref.at[...] if g == 0 else kv_all.at[16 - g]
        return pltpu.make_async_remote_copy(
            src_ref=half(src, part),
            dst_ref=half(kv_all.at[15 - g], part),
            send_sem=ccw_send_sems.at[g, part],
            recv_sem=ccw_recv_sems.at[g, part],
            device_id=(left,),
            device_id_type=pl.DeviceIdType.MESH,
        )

    cw = [[make_cw(h, p) for p in (0, 1)] for h in range(CW_HOPS)]
    ccw = [[make_ccw(g, p) for p in (0, 1)] for g in range(CCW_HOPS)]

    cw[0][0].start()
    ccw[0][0].start()
    cw[0][1].start()
    ccw[0][1].start()
    kv_all[0] = kv_ref[...]

    def process(slot, first):
        def bh_step(bh, _):
            q = q_ref[bh]
            k = kv_all[slot, 0, bh]
            v = kv_all[slot, 1, bh]
            s = lax.dot_general(
                q, k, (((1,), (1,)), ((), ())),
                preferred_element_type=jnp.float32,
            )
            m_blk = jnp.max(s, axis=-1, keepdims=True)
            if first:
                m_new = m_blk
                p = jnp.exp(s - m_new)
                l_ref[bh] = jnp.sum(p, axis=-1, keepdims=True)
                acc_ref[bh] = jnp.dot(
                    p.astype(jnp.bfloat16), v,
                    preferred_element_type=jnp.float32,
                )
            else:
                m_old = m_ref[bh]
                m_new = jnp.maximum(m_old, m_blk)
                alpha = jnp.exp(m_old - m_new)
                p = jnp.exp(s - m_new)
                l_ref[bh] = l_ref[bh] * alpha + jnp.sum(
                    p, axis=-1, keepdims=True
                )
                acc_ref[bh] = acc_ref[bh] * alpha + jnp.dot(
                    p.astype(jnp.bfloat16), v,
                    preferred_element_type=jnp.float32,
                )
            m_ref[bh] = m_new
            return _

        lax.fori_loop(0, BH, bh_step, 0)

    process(0, first=True)

    for step in range(1, CW_HOPS + 1):
        have_ccw = step <= CCW_HOPS
        cw[step - 1][0].wait_recv()
        if step < CW_HOPS:
            cw[step][0].start()
        if have_ccw:
            ccw[step - 1][0].wait_recv()
            if step < CCW_HOPS:
                ccw[step][0].start()
        cw[step - 1][1].wait_recv()
        if step < CW_HOPS:
            cw[step][1].start()
        if have_ccw:
            ccw[step - 1][1].wait_recv()
            if step < CCW_HOPS:
                ccw[step][1].start()
        process(step, first=False)
        if have_ccw:
            process(16 - step, first=False)

    for bh in range(BH):
        o_ref[bh] = acc_ref[bh] / l_ref[bh]

    for pair in cw + ccw:
        for rdma in pair:
            rdma.wait_send()


def kernel(Q, K, V):
    qt = (jnp.transpose(Q, (0, 2, 1, 3)).reshape(BH, S_LOC, D) * SCALE).astype(
        jnp.bfloat16
    )
    kt = jnp.transpose(K, (0, 2, 1, 3)).reshape(BH, S_LOC, D).astype(jnp.bfloat16)
    vt = jnp.transpose(V, (0, 2, 1, 3)).reshape(BH, S_LOC, D).astype(jnp.bfloat16)
    kv = jnp.stack([kt, vt])

    out = pl.pallas_call(
        _body,
        out_shape=jax.ShapeDtypeStruct((BH, S_LOC, D), jnp.float32),
        in_specs=[
            pl.BlockSpec(memory_space=pltpu.VMEM),
            pl.BlockSpec(memory_space=pltpu.VMEM),
        ],
        out_specs=pl.BlockSpec(memory_space=pltpu.VMEM),
        scratch_shapes=[
            pltpu.VMEM((N_DEV, 2, BH, S_LOC, D), jnp.bfloat16),
            pltpu.VMEM((BH, S_LOC, D), jnp.float32),
            pltpu.VMEM((BH, S_LOC, 1), jnp.float32),
            pltpu.VMEM((BH, S_LOC, 1), jnp.float32),
            pltpu.SemaphoreType.DMA((CW_HOPS, 2)),
            pltpu.SemaphoreType.DMA((CW_HOPS, 2)),
            pltpu.SemaphoreType.DMA((CCW_HOPS, 2)),
            pltpu.SemaphoreType.DMA((CCW_HOPS, 2)),
        ],
        compiler_params=pltpu.CompilerParams(
            collective_id=0, vmem_limit_bytes=100 * 1024 * 1024
        ),
    )(qt, kv)

    return jnp.transpose(out.reshape(2, 8, S_LOC, D), (0, 2, 1, 3))


# device time: 206902 ns/iter; 1.9918x vs baseline; 1.0154x over previous
import jax
import jax.numpy as jnp
from jax import lax
from jax.experimental import pallas as pl
from jax.experimental.pallas import tpu as pltpu

N_DEV = 16
S_LOC = 256
BH = 16
D = 64
SCALE = D ** -0.5


def _body(q_ref, kv_ref, o_ref, kv_all, acc_ref, m_ref, l_ref,
          z_send, z_recv, cw_send, cw_recv, ccw_send, ccw_recv):
    my = lax.axis_index("i")
    qpos = lax.rem(my, 4)
    base = my - qpos
    nxt = base + lax.rem(qpos + 1, 4)
    prv = base + lax.rem(qpos + 3, 4)
    up4 = lax.rem(my + 4, N_DEV)
    dn4 = lax.rem(my + 12, N_DEV)

    barrier_sem = pltpu.get_barrier_semaphore()
    for nbr in (nxt, prv, up4, dn4):
        pl.semaphore_signal(
            barrier_sem, inc=1,
            device_id=(nbr,), device_id_type=pl.DeviceIdType.MESH,
        )
    pl.semaphore_wait(barrier_sem, 4)

    def rdma(src_slot, dst_slot, tgt, send_sem, recv_sem):
        src = kv_ref.at[...] if src_slot == 0 else kv_all.at[src_slot]
        return pltpu.make_async_remote_copy(
            src_ref=src,
            dst_ref=kv_all.at[dst_slot],
            send_sem=send_sem,
            recv_sem=recv_sem,
            device_id=(tgt,),
            device_id_type=pl.DeviceIdType.MESH,
        )

    U0 = rdma(0, 12, up4, z_send.at[0], z_recv.at[0])
    U1 = rdma(12, 8, up4, z_send.at[1], z_recv.at[1])
    D0 = rdma(0, 4, dn4, z_send.at[2], z_recv.at[2])
    cw = [
        rdma(0, 3, nxt, cw_send.at[0], cw_recv.at[0]),
        rdma(4, 7, nxt, cw_send.at[1], cw_recv.at[1]),
        rdma(8, 11, nxt, cw_send.at[2], cw_recv.at[2]),
        rdma(12, 15, nxt, cw_send.at[3], cw_recv.at[3]),
        rdma(3, 2, nxt, cw_send.at[4], cw_recv.at[4]),
        rdma(7, 6, nxt, cw_send.at[5], cw_recv.at[5]),
    ]
    ccw = [
        rdma(0, 1, prv, ccw_send.at[0], ccw_recv.at[0]),
        rdma(4, 5, prv, ccw_send.at[1], ccw_recv.at[1]),
        rdma(8, 9, prv, ccw_send.at[2], ccw_recv.at[2]),
        rdma(12, 13, prv, ccw_send.at[3], ccw_recv.at[3]),
        rdma(9, 10, prv, ccw_send.at[4], ccw_recv.at[4]),
        rdma(13, 14, prv, ccw_send.at[5], ccw_recv.at[5]),
    ]

    def process(slot, first=False):
        def bh_step(bh, carry):
            q = q_ref[bh]
            k = kv_all[slot, 0, bh]
            v = kv_all[slot, 1, bh]
            s = lax.dot_general(
                q, k, (((1,), (1,)), ((), ())),
                preferred_element_type=jnp.float32,
            )
            m_blk = jnp.max(s, axis=-1, keepdims=True)
            if first:
                m_new = m_blk
                p = jnp.exp(s - m_new)
                l_ref[bh] = jnp.sum(p, axis=-1, keepdims=True)
                acc_ref[bh] = jnp.dot(
                    p.astype(jnp.bfloat16), v,
                    preferred_element_type=jnp.float32,
                )
            else:
                m_old = m_ref[bh]
                m_new = jnp.maximum(m_old, m_blk)
                alpha = jnp.exp(m_old - m_new)
                p = jnp.exp(s - m_new)
                l_ref[bh] = l_ref[bh] * alpha + jnp.sum(
                    p, axis=-1, keepdims=True
                )
                acc_ref[bh] = acc_ref[bh] * alpha + jnp.dot(
                    p.astype(jnp.bfloat16), v,
                    preferred_element_type=jnp.float32,
                )
            m_ref[bh] = m_new
            return carry

        lax.fori_loop(0, BH, bh_step, 0)

    U0.start()
    D0.start()
    cw[0].start()
    ccw[0].start()
    kv_all[0] = kv_ref[...]
    process(0, first=True)

    U0.wait_recv()
    U1.start()
    cw[3].start()
    ccw[3].start()
    process(12)
    D0.wait_recv()
    cw[1].start()
    ccw[1].start()
    process(4)
    U1.wait_recv()
    cw[2].start()
    ccw[2].start()
    process(8)

    cw[0].wait_recv()
    cw[4].start()
    process(3)
    ccw[0].wait_recv()
    process(1)
    cw[1].wait_recv()
    cw[5].start()
    process(7)
    ccw[1].wait_recv()
    process(5)
    cw[2].wait_recv()
    process(11)
    ccw[2].wait_recv()
    ccw[4].start()
    process(9)
    cw[3].wait_recv()
    process(15)
    ccw[3].wait_recv()
    ccw[5].start()
    process(13)
    cw[4].wait_recv()
    process(2)
    cw[5].wait_recv()
    process(6)
    ccw[4].wait_recv()
    process(10)
    ccw[5].wait_recv()
    process(14)

    for bh in range(BH):
        o_ref[bh] = acc_ref[bh] / l_ref[bh]

    for r in [U0, U1, D0] + cw + ccw:
        r.wait_send()


def kernel(Q, K, V):
    qt = (jnp.transpose(Q, (0, 2, 1, 3)).reshape(BH, S_LOC, D) * SCALE).astype(
        jnp.bfloat16
    )
    kt = jnp.transpose(K, (0, 2, 1, 3)).reshape(BH, S_LOC, D).astype(jnp.bfloat16)
    vt = jnp.transpose(V, (0, 2, 1, 3)).reshape(BH, S_LOC, D).astype(jnp.bfloat16)
    kv = jnp.stack([kt, vt])

    out = pl.pallas_call(
        _body,
        out_shape=jax.ShapeDtypeStruct((BH, S_LOC, D), jnp.float32),
        in_specs=[
            pl.BlockSpec(memory_space=pltpu.VMEM),
            pl.BlockSpec(memory_space=pltpu.VMEM),
        ],
        out_specs=pl.BlockSpec(memory_space=pltpu.VMEM),
        scratch_shapes=[
            pltpu.VMEM((N_DEV, 2, BH, S_LOC, D), jnp.bfloat16),
            pltpu.VMEM((BH, S_LOC, D), jnp.float32),
            pltpu.VMEM((BH, S_LOC, 1), jnp.float32),
            pltpu.VMEM((BH, S_LOC, 1), jnp.float32),
            pltpu.SemaphoreType.DMA((3,)),
            pltpu.SemaphoreType.DMA((3,)),
            pltpu.SemaphoreType.DMA((6,)),
            pltpu.SemaphoreType.DMA((6,)),
            pltpu.SemaphoreType.DMA((6,)),
            pltpu.SemaphoreType.DMA((6,)),
        ],
        compiler_params=pltpu.CompilerParams(
            collective_id=0, vmem_limit_bytes=100 * 1024 * 1024
        ),
    )(qt, kv)

    return jnp.transpose(out.reshape(2, 8, S_LOC, D), (0, 2, 1, 3))
